# Initial kernel scaffold; baseline (speedup 1.0000x reference)
import jax
import jax.numpy as jnp
from jax.experimental import pallas as pl


def kernel(case_idx, code_idx, case_y, code_y):
    raise NotImplementedError("write your pallas kernel here")


# speedup vs baseline: 3.5338x; 3.5338x over previous
import functools

import jax
import jax.numpy as jnp
from jax import lax
from jax.experimental import pallas as pl
from jax.experimental.pallas import tpu as pltpu
from jax.experimental.pallas import tpu_sc as plsc

_E = 262144
_NUM_CASES = 2048
_NUM_CODES = 8192
_NUM_NODES = _NUM_CASES + _NUM_CODES

_NC = 2
_NS = 16
_NW = _NC * _NS
_CPT = 52
_HWORDS = _CPT * _NUM_CASES
_NPASS = 5
_COV_CODES = _NPASS * _NW * _CPT
_KCHUNK = 8192
_NCHUNK = _E // _KCHUNK


def _key_body(case_ref, code_ref, k_ref):
    k_ref[...] = code_ref[...] * _NUM_CASES + case_ref[...]


def _make_keys(case_idx, code_idx):
    keys = pl.pallas_call(
        _key_body,
        out_shape=jax.ShapeDtypeStruct((256, 1024), jnp.int32),
    )(case_idx.reshape(256, 1024), code_idx.reshape(256, 1024))
    return keys.reshape(_E)


def _sc_body(keys_hbm, h_hbm, kbuf, hist):
    c = lax.axis_index("c")
    s = lax.axis_index("s")
    wid = s * _NC + c
    zeros16 = jnp.zeros((16,), jnp.int32)

    for p in range(_NPASS):
        code0 = p * (_NW * _CPT) + wid * _CPT
        base = code0 * _NUM_CASES

        def _z(i, carry):
            hist[pl.ds(i * 16, 16)] = zeros16
            return carry

        lax.fori_loop(0, _HWORDS // 16, _z, 0)

        def _chunk(ci, carry):
            pltpu.sync_copy(keys_hbm.at[pl.ds(ci * _KCHUNK, _KCHUNK)], kbuf)

            def _g(g, carry2):
                local = kbuf[pl.ds(g * 16, 16)] - base
                active = (local >= 0) & (local < _HWORDS)
                cnt, last = plsc.scan_count(local, mask=active)
                plsc.addupdate_scatter(
                    hist, [local], cnt, mask=last & active)
                return carry2

            lax.fori_loop(0, _KCHUNK // 16, _g, 0)
            return carry

        lax.fori_loop(0, _NCHUNK, _chunk, 0)

        pltpu.sync_copy(hist, h_hbm.at[pl.ds(base, _HWORDS)])


_sc_hist = functools.partial(
    pl.kernel,
    out_type=jax.ShapeDtypeStruct((_COV_CODES * _NUM_CASES,), jnp.int32),
    mesh=plsc.VectorSubcoreMesh(
        core_axis_name="c", subcore_axis_name="s", num_cores=_NC,
        num_subcores=_NS,
    ),
    scratch_types=[
        pltpu.VMEM((_KCHUNK,), jnp.int32),
        pltpu.VMEM((_HWORDS,), jnp.int32),
    ],
    compiler_params=pltpu.CompilerParams(needs_layout_passes=False),
)(_sc_body)


def _tc_body(h_ref, cy_ref, ay_ref, o_ref):
    i = pl.program_id(0)

    @pl.when(i < _NUM_CODES // 512)
    def _():
        h = h_ref[...].astype(jnp.float32)
        both = cy_ref[...] * ay_ref[...]
        o_ref[...] = jnp.where(
            both == 1.0, jnp.where(h > 0.0, -1.0, 0.0), h
        )

    @pl.when(i >= _NUM_CODES // 512)
    def _():
        o_ref[...] = jnp.zeros_like(o_ref)


def _tc_transform(h, cy, ay):
    nbk = _NUM_NODES // 512
    hbk = _NUM_CODES // 512
    return pl.pallas_call(
        _tc_body,
        out_shape=jax.ShapeDtypeStruct((_NUM_NODES, _NUM_CASES), jnp.float32),
        grid=(nbk,),
        in_specs=[
            pl.BlockSpec((512, _NUM_CASES),
                         lambda i: (jnp.minimum(i, hbk - 1), 0)),
            pl.BlockSpec((512, 1), lambda i: (jnp.minimum(i, hbk - 1), 0)),
            pl.BlockSpec((1, _NUM_CASES), lambda i: (0, 0)),
        ],
        out_specs=pl.BlockSpec((512, _NUM_CASES), lambda i: (i, 0)),
    )(h, cy, ay)


def kernel(case_idx, code_idx, case_y, code_y):
    keys = _make_keys(case_idx, code_idx)
    h = _sc_hist(keys)
    h = h.reshape(_COV_CODES, _NUM_CASES)
    cy = code_y.astype(jnp.float32).reshape(_NUM_CODES, 1)
    ay = case_y.astype(jnp.float32).reshape(1, _NUM_CASES)
    return _tc_transform(h, cy, ay)


# speedup vs baseline: 9.2793x; 2.6259x over previous
import functools

import jax
import jax.numpy as jnp
from jax import lax
from jax.experimental import pallas as pl
from jax.experimental.pallas import tpu as pltpu
from jax.experimental.pallas import tpu_sc as plsc

_E = 262144
_NUM_CASES = 2048
_NUM_CODES = 8192
_NUM_NODES = _NUM_CASES + _NUM_CODES

_NC = 2
_NS = 16
_NW = _NC * _NS
_CPT = 52
_HWORDS = _CPT * _NUM_CASES
_NPASS = 5
_COV_CODES = _NPASS * _NW * _CPT
_KCHUNK = 8192
_NCHUNK = _E // _KCHUNK


def _key_body(case_ref, code_ref, k_ref):
    k_ref[...] = code_ref[...] * _NUM_CASES + case_ref[...]


def _make_keys(case_idx, code_idx):
    keys = pl.pallas_call(
        _key_body,
        out_shape=jax.ShapeDtypeStruct((256, 1024), jnp.int32),
    )(case_idx.reshape(256, 1024), code_idx.reshape(256, 1024))
    return keys.reshape(_E)


def _sc_body(keys_hbm, h_hbm, kbuf, hist):
    c = lax.axis_index("c")
    s = lax.axis_index("s")
    wid = s * _NC + c
    zeros16 = jnp.zeros((16,), jnp.int32)

    for p in range(_NPASS):
        code0 = p * (_NW * _CPT) + wid * _CPT
        base = code0 * _NUM_CASES

        def _z(i, carry):
            for u in range(8):
                hist[pl.ds(i * 128 + u * 16, 16)] = zeros16
            return carry

        lax.fori_loop(0, _HWORDS // 128, _z, 0)

        def _chunk(ci, carry):
            pltpu.sync_copy(keys_hbm.at[pl.ds(ci * _KCHUNK, _KCHUNK)], kbuf)

            def _g(g, carry2):
                locs, cnts, lasts = [], [], []
                for u in range(4):
                    local = kbuf[pl.ds(g * 64 + u * 16, 16)] - base
                    active = (local >= 0) & (local < _HWORDS)
                    cnt, last = plsc.scan_count(local, mask=active)
                    locs.append(local)
                    cnts.append(cnt)
                    lasts.append(last & active)
                for u in range(4):
                    plsc.addupdate_scatter(
                        hist, [locs[u]], cnts[u], mask=lasts[u])
                return carry2

            lax.fori_loop(0, _KCHUNK // 64, _g, 0)
            return carry

        lax.fori_loop(0, _NCHUNK, _chunk, 0)

        pltpu.sync_copy(hist, h_hbm.at[pl.ds(base, _HWORDS)])


_sc_hist = functools.partial(
    pl.kernel,
    out_type=jax.ShapeDtypeStruct((_COV_CODES * _NUM_CASES,), jnp.int32),
    mesh=plsc.VectorSubcoreMesh(
        core_axis_name="c", subcore_axis_name="s", num_cores=_NC,
        num_subcores=_NS,
    ),
    scratch_types=[
        pltpu.VMEM((_KCHUNK,), jnp.int32),
        pltpu.VMEM((_HWORDS,), jnp.int32),
    ],
    compiler_params=pltpu.CompilerParams(needs_layout_passes=False),
)(_sc_body)


def _tc_body(h_ref, cy_ref, ay_ref, o_ref):
    i = pl.program_id(0)

    @pl.when(i < _NUM_CODES // 512)
    def _():
        h = h_ref[...].astype(jnp.float32)
        both = cy_ref[...] * ay_ref[...]
        o_ref[...] = jnp.where(
            both == 1.0, jnp.where(h > 0.0, -1.0, 0.0), h
        )

    @pl.when(i >= _NUM_CODES // 512)
    def _():
        o_ref[...] = jnp.zeros_like(o_ref)


def _tc_transform(h, cy, ay):
    nbk = _NUM_NODES // 512
    hbk = _NUM_CODES // 512
    return pl.pallas_call(
        _tc_body,
        out_shape=jax.ShapeDtypeStruct((_NUM_NODES, _NUM_CASES), jnp.float32),
        grid=(nbk,),
        in_specs=[
            pl.BlockSpec((512, _NUM_CASES),
                         lambda i: (jnp.minimum(i, hbk - 1), 0)),
            pl.BlockSpec((512, 1), lambda i: (jnp.minimum(i, hbk - 1), 0)),
            pl.BlockSpec((1, _NUM_CASES), lambda i: (0, 0)),
        ],
        out_specs=pl.BlockSpec((512, _NUM_CASES), lambda i: (i, 0)),
    )(h, cy, ay)


def kernel(case_idx, code_idx, case_y, code_y):
    keys = _make_keys(case_idx, code_idx)
    h = _sc_hist(keys)
    h = h.reshape(_COV_CODES, _NUM_CASES)
    cy = code_y.astype(jnp.float32).reshape(_NUM_CODES, 1)
    ay = case_y.astype(jnp.float32).reshape(1, _NUM_CASES)
    return _tc_transform(h, cy, ay)


# speedup vs baseline: 11.9679x; 1.2897x over previous
import functools

import jax
import jax.numpy as jnp
from jax import lax
from jax.experimental import pallas as pl
from jax.experimental.pallas import tpu as pltpu
from jax.experimental.pallas import tpu_sc as plsc

_E = 262144
_NUM_CASES = 2048
_NUM_CODES = 8192
_NUM_NODES = _NUM_CASES + _NUM_CODES

_NC = 2
_NS = 16
_NW = _NC * _NS
_CPT = 52
_HWORDS = _CPT * _NUM_CASES
_NPASS = 5
_COV_CODES = _NPASS * _NW * _CPT
_KCHUNK = 8192
_NCHUNK = _E // _KCHUNK


def _key_body(case_ref, code_ref, k_ref):
    k_ref[...] = code_ref[...] * _NUM_CASES + case_ref[...]


def _make_keys(case_idx, code_idx):
    keys = pl.pallas_call(
        _key_body,
        out_shape=jax.ShapeDtypeStruct((256, 1024), jnp.int32),
    )(case_idx.reshape(256, 1024), code_idx.reshape(256, 1024))
    return keys.reshape(_E)


def _sc_body(keys_hbm, h_hbm, kbuf, hist):
    c = lax.axis_index("c")
    s = lax.axis_index("s")
    wid = s * _NC + c
    zeros16 = jnp.zeros((16,), jnp.int32)

    for p in range(_NPASS):
        code0 = p * (_NW * _CPT) + wid * _CPT
        base = code0 * _NUM_CASES

        def _z(i, carry):
            for u in range(8):
                hist[pl.ds(i * 128 + u * 16, 16)] = zeros16
            return carry

        lax.fori_loop(0, _HWORDS // 128, _z, 0)

        def _chunk(ci, carry):
            pltpu.sync_copy(keys_hbm.at[pl.ds(ci * _KCHUNK, _KCHUNK)], kbuf)

            def _g(g, carry2):
                locs, cnts, lasts = [], [], []
                for u in range(8):
                    local = kbuf[pl.ds(g * 128 + u * 16, 16)] - base
                    active = (local >= 0) & (local < _HWORDS)
                    cnt, last = plsc.scan_count(local, mask=active)
                    locs.append(local)
                    cnts.append(cnt)
                    lasts.append(last & active)
                for u in range(8):
                    plsc.addupdate_scatter(
                        hist, [locs[u]], cnts[u], mask=lasts[u])
                return carry2

            lax.fori_loop(0, _KCHUNK // 128, _g, 0)
            return carry

        lax.fori_loop(0, _NCHUNK, _chunk, 0)

        pltpu.sync_copy(hist, h_hbm.at[pl.ds(base, _HWORDS)])


_sc_hist = functools.partial(
    pl.kernel,
    out_type=jax.ShapeDtypeStruct((_COV_CODES * _NUM_CASES,), jnp.int32),
    mesh=plsc.VectorSubcoreMesh(
        core_axis_name="c", subcore_axis_name="s", num_cores=_NC,
        num_subcores=_NS,
    ),
    scratch_types=[
        pltpu.VMEM((_KCHUNK,), jnp.int32),
        pltpu.VMEM((_HWORDS,), jnp.int32),
    ],
    compiler_params=pltpu.CompilerParams(needs_layout_passes=False),
)(_sc_body)


def _tc_body(h_ref, cy_ref, ay_ref, o_ref):
    i = pl.program_id(0)

    @pl.when(i < _NUM_CODES // 512)
    def _():
        h = h_ref[...].astype(jnp.float32)
        both = cy_ref[...] * ay_ref[...]
        o_ref[...] = jnp.where(
            both == 1.0, jnp.where(h > 0.0, -1.0, 0.0), h
        )

    @pl.when(i >= _NUM_CODES // 512)
    def _():
        o_ref[...] = jnp.zeros_like(o_ref)


def _tc_transform(h, cy, ay):
    nbk = _NUM_NODES // 512
    hbk = _NUM_CODES // 512
    return pl.pallas_call(
        _tc_body,
        out_shape=jax.ShapeDtypeStruct((_NUM_NODES, _NUM_CASES), jnp.float32),
        grid=(nbk,),
        in_specs=[
            pl.BlockSpec((512, _NUM_CASES),
                         lambda i: (jnp.minimum(i, hbk - 1), 0)),
            pl.BlockSpec((512, 1), lambda i: (jnp.minimum(i, hbk - 1), 0)),
            pl.BlockSpec((1, _NUM_CASES), lambda i: (0, 0)),
        ],
        out_specs=pl.BlockSpec((512, _NUM_CASES), lambda i: (i, 0)),
    )(h, cy, ay)


def kernel(case_idx, code_idx, case_y, code_y):
    keys = _make_keys(case_idx, code_idx)
    h = _sc_hist(keys)
    h = h.reshape(_COV_CODES, _NUM_CASES)
    cy = code_y.astype(jnp.float32).reshape(_NUM_CODES, 1)
    ay = case_y.astype(jnp.float32).reshape(1, _NUM_CASES)
    return _tc_transform(h, cy, ay)
